# Initial kernel scaffold; baseline (speedup 1.0000x reference)
import jax
import jax.numpy as jnp
from jax.experimental import pallas as pl


def kernel(input_vector, table):
    raise NotImplementedError("write your pallas kernel here")


# speedup vs baseline: 4.0821x; 4.0821x over previous
import functools

import jax
import jax.numpy as jnp
from jax import lax
from jax.experimental import pallas as pl
from jax.experimental.pallas import tpu as pltpu
from jax.experimental.pallas import tpu_sc as plsc

B0, B1 = 4096, 50
D = 64
B_TOTAL = B0 * B1
CHUNK = 128
NC, NS = 2, 16
NW = NC * NS
CHUNKS_PER_W = B_TOTAL // (CHUNK * NW)

_mesh = plsc.VectorSubcoreMesh(core_axis_name="c", subcore_axis_name="s")


@functools.partial(
    pl.kernel,
    mesh=_mesh,
    out_type=jax.ShapeDtypeStruct((B_TOTAL, D), jnp.float32),
    scratch_types=[
        pltpu.VMEM((CHUNKS_PER_W, CHUNK), jnp.int32),
        pltpu.VMEM((CHUNK, D), jnp.float32),
        pltpu.SemaphoreType.DMA,
    ],
    compiler_params=pltpu.CompilerParams(use_tc_tiling_on_sc=False),
)
def _gather_kernel(idx_hbm, table_hbm, out_hbm, idx_v, rows_v, sem):
    wid = lax.axis_index("s") * NC + lax.axis_index("c")
    row_base = wid * CHUNKS_PER_W
    pltpu.sync_copy(idx_hbm.at[wid], idx_v)

    def body(j, carry):
        pltpu.async_copy(table_hbm.at[idx_v.at[j]], rows_v, sem).wait()
        pltpu.sync_copy(
            rows_v, out_hbm.at[pl.ds((row_base + j) * CHUNK, CHUNK)]
        )
        return carry

    lax.fori_loop(0, CHUNKS_PER_W, body, 0)


def kernel(input_vector, table):
    idx = input_vector.astype(jnp.int32).reshape(NW, CHUNKS_PER_W, CHUNK)
    out = _gather_kernel(idx, table)
    return out.reshape(B0, B1, D)


# speedup vs baseline: 4.6524x; 1.1397x over previous
import functools

import jax
import jax.numpy as jnp
from jax import lax
from jax.experimental import pallas as pl
from jax.experimental.pallas import tpu as pltpu
from jax.experimental.pallas import tpu_sc as plsc

B0, B1 = 4096, 50
D = 64
B_TOTAL = B0 * B1
CHUNK = 128
NC, NS = 2, 16
NW = NC * NS
CHUNKS_PER_W = B_TOTAL // (CHUNK * NW)
GROUP = 5
NGROUPS = CHUNKS_PER_W // GROUP
GROUP_ROWS = GROUP * CHUNK

_mesh = plsc.VectorSubcoreMesh(core_axis_name="c", subcore_axis_name="s")


@functools.partial(
    pl.kernel,
    mesh=_mesh,
    out_type=jax.ShapeDtypeStruct((B_TOTAL, D), jnp.float32),
    scratch_types=[
        pltpu.VMEM((CHUNKS_PER_W, CHUNK), jnp.int32),
        pltpu.VMEM((GROUP_ROWS, D), jnp.float32),
        pltpu.VMEM((GROUP_ROWS, D), jnp.float32),
        pltpu.SemaphoreType.DMA,
        pltpu.SemaphoreType.DMA,
        pltpu.SemaphoreType.DMA,
        pltpu.SemaphoreType.DMA,
    ],
    compiler_params=pltpu.CompilerParams(use_tc_tiling_on_sc=False),
)
def _gather_kernel(idx_hbm, table_hbm, out_hbm, idx_v, rows0, rows1,
                   gsem0, gsem1, osem0, osem1):
    wid = lax.axis_index("s") * NC + lax.axis_index("c")
    row_base = wid * CHUNKS_PER_W * CHUNK
    pltpu.sync_copy(idx_hbm.at[wid], idx_v)

    rows = (rows0, rows1)
    gsem = (gsem0, gsem1)
    osem = (osem0, osem1)

    def fire_group(g, buf, sem):
        for c in range(GROUP):
            pltpu.async_copy(
                table_hbm.at[idx_v.at[g * GROUP + c]],
                buf.at[pl.ds(c * CHUNK, CHUNK)],
                sem,
            )

    def drain_group(buf, sem):
        pltpu.make_async_copy(
            table_hbm.at[pl.ds(0, GROUP_ROWS)], buf, sem
        ).wait()

    def wait_out(buf, sem):
        pltpu.make_async_copy(
            buf, out_hbm.at[pl.ds(0, GROUP_ROWS)], sem
        ).wait()

    fire_group(0, rows0, gsem0)

    def step(g2, carry):
        for p in range(2):
            g = g2 * 2 + p
            q = 1 - p

            @pl.when(g + 1 < NGROUPS)
            def _():
                @pl.when(g >= 1)
                def _():
                    wait_out(rows[q], osem[q])
                fire_group(g + 1, rows[q], gsem[q])

            drain_group(rows[p], gsem[p])
            pltpu.make_async_copy(
                rows[p],
                out_hbm.at[pl.ds(row_base + g * GROUP_ROWS, GROUP_ROWS)],
                osem[p],
            ).start()
        return carry

    lax.fori_loop(0, NGROUPS // 2, step, 0)
    wait_out(rows0, osem0)
    wait_out(rows1, osem1)


def kernel(input_vector, table):
    idx = input_vector.astype(jnp.int32).reshape(NW, CHUNKS_PER_W, CHUNK)
    out = _gather_kernel(idx, table)
    return out.reshape(B0, B1, D)


# speedup vs baseline: 4.6695x; 1.0037x over previous
import functools

import jax
import jax.numpy as jnp
from jax import lax
from jax.experimental import pallas as pl
from jax.experimental.pallas import tpu as pltpu
from jax.experimental.pallas import tpu_sc as plsc

B0, B1 = 4096, 50
D = 64
B_TOTAL = B0 * B1
CHUNK = 128
NC, NS = 2, 16
NW = NC * NS
CHUNKS_PER_W = B_TOTAL // (CHUNK * NW)
GROUP = 5
NGROUPS = CHUNKS_PER_W // GROUP
GROUP_ROWS = GROUP * CHUNK

_mesh = plsc.VectorSubcoreMesh(core_axis_name="c", subcore_axis_name="s")


@functools.partial(
    pl.kernel,
    mesh=_mesh,
    out_type=jax.ShapeDtypeStruct((B_TOTAL, D), jnp.float32),
    scratch_types=[
        pltpu.VMEM((NGROUPS, GROUP_ROWS), jnp.int32),
        pltpu.VMEM((GROUP_ROWS, D), jnp.float32),
        pltpu.VMEM((GROUP_ROWS, D), jnp.float32),
        pltpu.SemaphoreType.DMA,
        pltpu.SemaphoreType.DMA,
        pltpu.SemaphoreType.DMA,
        pltpu.SemaphoreType.DMA,
    ],
    compiler_params=pltpu.CompilerParams(use_tc_tiling_on_sc=False),
)
def _gather_kernel(idx_hbm, table_hbm, out_hbm, idx_v, rows0, rows1,
                   gsem0, gsem1, osem0, osem1):
    wid = lax.axis_index("s") * NC + lax.axis_index("c")
    row_base = wid * CHUNKS_PER_W * CHUNK
    pltpu.sync_copy(idx_hbm.at[wid], idx_v)

    rows = (rows0, rows1)
    gsem = (gsem0, gsem1)
    osem = (osem0, osem1)

    def fire_group(g, buf, sem):
        pltpu.async_copy(
            table_hbm.at[idx_v.at[g]],
            buf,
            sem,
        )

    def drain_group(buf, sem):
        pltpu.make_async_copy(
            table_hbm.at[pl.ds(0, GROUP_ROWS)], buf, sem
        ).wait()

    def wait_out(buf, sem):
        pltpu.make_async_copy(
            buf, out_hbm.at[pl.ds(0, GROUP_ROWS)], sem
        ).wait()

    fire_group(0, rows0, gsem0)

    def step(g2, carry):
        for p in range(2):
            g = g2 * 2 + p
            q = 1 - p

            @pl.when(g + 1 < NGROUPS)
            def _():
                @pl.when(g >= 1)
                def _():
                    wait_out(rows[q], osem[q])
                fire_group(g + 1, rows[q], gsem[q])

            drain_group(rows[p], gsem[p])
            pltpu.make_async_copy(
                rows[p],
                out_hbm.at[pl.ds(row_base + g * GROUP_ROWS, GROUP_ROWS)],
                osem[p],
            ).start()
        return carry

    lax.fori_loop(0, NGROUPS // 2, step, 0)
    wait_out(rows0, osem0)
    wait_out(rows1, osem1)


def kernel(input_vector, table):
    idx = input_vector.astype(jnp.int32).reshape(NW, NGROUPS, GROUP_ROWS)
    out = _gather_kernel(idx, table)
    return out.reshape(B0, B1, D)


# speedup vs baseline: 5.3028x; 1.1356x over previous
import functools

import jax
import jax.numpy as jnp
from jax import lax
from jax.experimental import pallas as pl
from jax.experimental.pallas import tpu as pltpu
from jax.experimental.pallas import tpu_sc as plsc

B0, B1 = 4096, 50
D = 64
CHUNK = 128
NC, NS = 2, 16
NW = NC * NS
S_STRIDE = 129

_mesh = plsc.VectorSubcoreMesh(core_axis_name="c", subcore_axis_name="s")


@functools.partial(
    pl.kernel,
    mesh=_mesh,
    out_type=jax.ShapeDtypeStruct((B1, 8, NW, 8, CHUNK), jnp.float32),
    scratch_types=[
        pltpu.VMEM((B1, CHUNK), jnp.int32),
        pltpu.VMEM((CHUNK, D), jnp.float32),
        pltpu.VMEM((CHUNK, D), jnp.float32),
        pltpu.VMEM((D, S_STRIDE), jnp.float32),
        pltpu.VMEM((D, S_STRIDE), jnp.float32),
        pltpu.SemaphoreType.DMA,
        pltpu.SemaphoreType.DMA,
        pltpu.SemaphoreType.DMA,
        pltpu.SemaphoreType.DMA,
    ],
    compiler_params=pltpu.CompilerParams(use_tc_tiling_on_sc=False, needs_layout_passes=False),
)
def _gather_kernel(idx_hbm, table_hbm, out_hbm, idx_v, rows0, rows1,
                   stage0, stage1, gsem0, gsem1, osem0, osem1):
    w = lax.axis_index("s") * NC + lax.axis_index("c")
    pltpu.sync_copy(idx_hbm.at[w], idx_v)

    rows = (rows0, rows1)
    stage = (stage0, stage1)
    gsem = (gsem0, gsem1)
    osem = (osem0, osem1)

    iota = lax.broadcasted_iota(jnp.int32, (16,), 0)
    d_idx = [iota + (q * 16) for q in range(4)]

    def fire_gather(s, p):
        pltpu.async_copy(table_hbm.at[idx_v.at[s]], rows[p], gsem[p])

    def wait_gather(p):
        pltpu.make_async_copy(
            table_hbm.at[pl.ds(0, CHUNK)], rows[p], gsem[p]
        ).wait()

    def wait_outs(p):
        for dt in range(8):
            pltpu.make_async_copy(
                stage[p].at[pl.ds(dt * 8, 8), pl.ds(0, CHUNK)],
                out_hbm.at[0, dt, 0],
                osem[p],
            ).wait()

    def transpose_unit(p):
        for b in range(CHUNK):
            b_bcast = jnp.full((16,), b, jnp.int32)
            for q in range(4):
                v = rows[p][b, pl.ds(q * 16, 16)]
                plsc.store_scatter(stage[p], [d_idx[q], b_bcast], v)

    def fire_outs(s, p):
        for dt in range(8):
            pltpu.async_copy(
                stage[p].at[pl.ds(dt * 8, 8), pl.ds(0, CHUNK)],
                out_hbm.at[s, dt, w],
                osem[p],
            )

    fire_gather(0, 0)

    def step(s2, carry):
        for p in range(2):
            s = s2 * 2 + p

            @pl.when(s + 1 < B1)
            def _():
                fire_gather(s + 1, 1 - p)

            wait_gather(p)

            @pl.when(s >= 2)
            def _():
                wait_outs(p)

            transpose_unit(p)
            fire_outs(s, p)
        return carry

    lax.fori_loop(0, B1 // 2, step, 0)
    wait_outs(0)
    wait_outs(1)


def kernel(input_vector, table):
    ivt = input_vector.T.astype(jnp.int32)
    idx_op = ivt.reshape(B1, NW, CHUNK).transpose(1, 0, 2)
    o5 = _gather_kernel(idx_op, table)
    return o5.transpose(2, 4, 0, 1, 3).reshape(B0, B1, D)
